# baseline (speedup 1.0000x reference)
import functools

import jax
import jax.numpy as jnp
from jax import lax
from jax.experimental import pallas as pl
from jax.experimental.pallas import tpu as pltpu
from jax.experimental.pallas import tpu_sc as plsc

_N = 10000
_D = 128
_E = 320000
_NW = 32
_EPW = _E // _NW
_G = 80
_NG = _EPW // _G


def _mlp_block(x_ref, w_ref, b_ref, o_ref):
    y = jnp.dot(x_ref[...], w_ref[...], preferred_element_type=jnp.float32)
    o_ref[...] = jnp.maximum(y + b_ref[...], 0.0)


def _transform(x, w, b, bl=2000):
    n, d = x.shape
    return pl.pallas_call(
        _mlp_block,
        grid=(n // bl,),
        in_specs=[
            pl.BlockSpec((bl, d), lambda i: (i, 0)),
            pl.BlockSpec((d, d), lambda i: (0, 0)),
            pl.BlockSpec((1, d), lambda i: (0, 0)),
        ],
        out_specs=pl.BlockSpec((bl, d), lambda i: (i, 0)),
        out_shape=jax.ShapeDtypeStruct((n, d), jnp.float32),
    )(x, w, b.reshape(1, d))


def _edge_values(u_tab, v_tab, src, dst, ev):
    mesh = plsc.VectorSubcoreMesh(core_axis_name="c", subcore_axis_name="s")

    @functools.partial(
        pl.kernel,
        mesh=mesh,
        out_type=jax.ShapeDtypeStruct((_E,), jnp.float32),
        compiler_params=pltpu.CompilerParams(needs_layout_passes=False),
        scratch_types=[
            pltpu.VMEM((_EPW,), jnp.int32),
            pltpu.VMEM((_EPW,), jnp.int32),
            pltpu.VMEM((_EPW,), jnp.float32),
            pltpu.VMEM((_EPW,), jnp.float32),
            pltpu.VMEM((2, _G, _D), jnp.float32),
            pltpu.VMEM((2, _G, _D), jnp.float32),
            pltpu.SemaphoreType.DMA,
            pltpu.SemaphoreType.DMA,
            pltpu.SemaphoreType.DMA,
            pltpu.SemaphoreType.DMA,
        ],
    )
    def body(u_hbm, v_hbm, src_hbm, dst_hbm, ev_hbm, out_hbm,
             src_v, dst_v, ev_v, out_v, u_rows, v_rows,
             sem_u0, sem_v0, sem_u1, sem_v1):
        wid = lax.axis_index("s") * 2 + lax.axis_index("c")
        base = wid * _EPW
        pltpu.sync_copy(src_hbm.at[pl.ds(base, _EPW)], src_v)
        pltpu.sync_copy(dst_hbm.at[pl.ds(base, _EPW)], dst_v)
        pltpu.sync_copy(ev_hbm.at[pl.ds(base, _EPW)], ev_v)

        lanes = lax.iota(jnp.int32, 16)
        sems = ((sem_u0, sem_v0), (sem_u1, sem_v1))

        def issue(g, b):
            gb = g * _G
            pltpu.async_copy(u_hbm.at[src_v.at[pl.ds(gb, _G)]],
                             u_rows.at[b], sems[b][0])
            pltpu.async_copy(v_hbm.at[dst_v.at[pl.ds(gb, _G)]],
                             v_rows.at[b], sems[b][1])

        def compute(g, b):
            gb = g * _G
            pltpu.make_async_copy(u_hbm.at[src_v.at[pl.ds(gb, _G)]],
                                  u_rows.at[b], sems[b][0]).wait()
            pltpu.make_async_copy(v_hbm.at[dst_v.at[pl.ds(gb, _G)]],
                                  v_rows.at[b], sems[b][1]).wait()

            def subgroup(sg, c):
                sgb = sg * 16
                d2 = jnp.zeros((16,), jnp.float32)
                for k in range(16):
                    e = sgb + k
                    acc = jnp.zeros((16,), jnp.float32)
                    for j in range(_D // 16):
                        du = (u_rows[b, e, pl.ds(j * 16, 16)]
                              - v_rows[b, e, pl.ds(j * 16, 16)])
                        acc = acc + du * du
                    d2 = jnp.where(lanes == k, jnp.sum(acc), d2)
                d2c = jnp.maximum(d2, 1e-30)
                bi = lax.bitcast_convert_type(d2c, jnp.int32)
                bi = 0x5F3759DF - lax.shift_right_arithmetic(bi, 1)
                y = lax.bitcast_convert_type(bi, jnp.float32)
                for _ in range(3):
                    y = y * (1.5 - 0.5 * d2c * y * y)
                dist = d2 * y
                sim = jnp.exp(dist)
                sig = 1.0 / (1.0 + jnp.exp(-sim))
                eb = gb + sgb
                out_v[pl.ds(eb, 16)] = ev_v[pl.ds(eb, 16)] * sig
                return c

            lax.fori_loop(0, _G // 16, subgroup, 0, unroll=True)

        issue(0, 0)

        def outer(tt, carry):
            g0 = tt * 2
            issue(g0 + 1, 1)
            compute(g0, 0)
            issue(g0 + 2, 0)
            compute(g0 + 1, 1)
            return carry

        lax.fori_loop(0, (_NG - 1) // 2, outer, 0)
        compute(_NG - 1, 0)
        pltpu.sync_copy(out_v, out_hbm.at[pl.ds(base, _EPW)])

    return body(u_tab, v_tab, src, dst, ev)


def kernel(Eu, Ev, W1, b1, W2, b2, edge_index, edge_val):
    u = _transform(Eu, W1, b1)
    v = _transform(Ev, W2, b2)
    return _edge_values(u, v, edge_index[0], edge_index[1], edge_val)


# speedup vs baseline: 2.1187x; 2.1187x over previous
import functools

import jax
import jax.numpy as jnp
from jax import lax
from jax.experimental import pallas as pl
from jax.experimental.pallas import tpu as pltpu
from jax.experimental.pallas import tpu_sc as plsc

_N = 10000
_D = 128
_E = 320000
_NW = 32
_EPW = _E // _NW
_G = 80
_NG = _EPW // _G


def _mlp_block_pos(x_ref, w_ref, b_ref, o_ref):
    y = jnp.dot(x_ref[...], w_ref[...], preferred_element_type=jnp.float32)
    o_ref[...] = jnp.maximum(y + b_ref[...], 0.0)


def _mlp_block_neg(x_ref, w_ref, b_ref, o_ref):
    y = jnp.dot(x_ref[...], w_ref[...], preferred_element_type=jnp.float32)
    o_ref[...] = jnp.minimum(-y - b_ref[...], 0.0)


def _transform(x, w, b, body, bl=2000):
    n, d = x.shape
    return pl.pallas_call(
        body,
        grid=(n // bl,),
        in_specs=[
            pl.BlockSpec((bl, d), lambda i: (i, 0)),
            pl.BlockSpec((d, d), lambda i: (0, 0)),
            pl.BlockSpec((1, d), lambda i: (0, 0)),
        ],
        out_specs=pl.BlockSpec((bl, d), lambda i: (i, 0)),
        out_shape=jax.ShapeDtypeStruct((n, d), jnp.float32),
    )(x, w, b.reshape(1, d))


def _edge_values(u_tab, vneg_tab, src, dst, ev):
    mesh = plsc.VectorSubcoreMesh(core_axis_name="c", subcore_axis_name="s")

    @functools.partial(
        pl.kernel,
        mesh=mesh,
        out_type=jax.ShapeDtypeStruct((_E,), jnp.float32),
        compiler_params=pltpu.CompilerParams(needs_layout_passes=False),
        scratch_types=[
            pltpu.VMEM((_EPW,), jnp.int32),
            pltpu.VMEM((_EPW,), jnp.int32),
            pltpu.VMEM((_EPW,), jnp.float32),
            pltpu.VMEM((_EPW,), jnp.float32),
            pltpu.VMEM((3, _G, _D), jnp.float32),
            pltpu.SemaphoreType.DMA,
            pltpu.SemaphoreType.DMA,
            pltpu.SemaphoreType.DMA,
            pltpu.SemaphoreType.DMA,
            pltpu.SemaphoreType.DMA,
            pltpu.SemaphoreType.DMA,
        ],
    )
    def body(u_hbm, v_hbm, src_hbm, dst_hbm, ev_hbm, out_hbm,
             src_v, dst_v, ev_v, out_v, du_rows,
             su0, su1, su2, sv0, sv1, sv2):
        wid = lax.axis_index("s") * 2 + lax.axis_index("c")
        base = wid * _EPW
        pltpu.sync_copy(src_hbm.at[pl.ds(base, _EPW)], src_v)
        pltpu.sync_copy(dst_hbm.at[pl.ds(base, _EPW)], dst_v)
        pltpu.sync_copy(ev_hbm.at[pl.ds(base, _EPW)], ev_v)

        lanes = lax.iota(jnp.int32, 16)
        sem_u = (su0, su1, su2)
        sem_v = (sv0, sv1, sv2)

        def issue_u(g, b):
            pltpu.async_copy(u_hbm.at[src_v.at[pl.ds(g * _G, _G)]],
                             du_rows.at[b], sem_u[b])

        def wait_u(g, b):
            pltpu.make_async_copy(u_hbm.at[src_v.at[pl.ds(g * _G, _G)]],
                                  du_rows.at[b], sem_u[b]).wait()

        def issue_vadd(g, b):
            pltpu.async_copy(v_hbm.at[dst_v.at[pl.ds(g * _G, _G)]],
                             du_rows.at[b], sem_v[b], add=True)

        def wait_vadd(g, b):
            pltpu.make_async_copy(v_hbm.at[dst_v.at[pl.ds(g * _G, _G)]],
                                  du_rows.at[b], sem_v[b]).wait()

        def compute(g, b):
            gb = g * _G

            def subgroup(sg, c):
                sgb = sg * 16
                d2 = jnp.zeros((16,), jnp.float32)
                for k in range(16):
                    e = sgb + k
                    acc = jnp.zeros((16,), jnp.float32)
                    for j in range(_D // 16):
                        du = du_rows[b, e, pl.ds(j * 16, 16)]
                        acc = acc + du * du
                    d2 = jnp.where(lanes == k, jnp.sum(acc), d2)
                d2c = jnp.maximum(d2, 1e-30)
                bi = lax.bitcast_convert_type(d2c, jnp.int32)
                bi = 0x5F3759DF - lax.shift_right_arithmetic(bi, 1)
                y = lax.bitcast_convert_type(bi, jnp.float32)
                for _ in range(3):
                    y = y * (1.5 - 0.5 * d2c * y * y)
                dist = d2 * y
                sim = jnp.exp(dist)
                sig = 1.0 / (1.0 + jnp.exp(-sim))
                eb = gb + sgb
                out_v[pl.ds(eb, 16)] = ev_v[pl.ds(eb, 16)] * sig
                return c

            lax.fori_loop(0, _G // 16, subgroup, 0)

        issue_u(0, 0)
        wait_u(0, 0)
        issue_vadd(0, 0)
        issue_u(1, 1)

        def outer(tt, carry):
            g0 = tt * 3
            for k in range(3):
                g = g0 + k
                b = k
                issue_u(g + 2, (b + 2) % 3)
                wait_u(g + 1, (b + 1) % 3)
                issue_vadd(g + 1, (b + 1) % 3)
                wait_vadd(g, b)
                compute(g, b)
            return carry

        lax.fori_loop(0, (_NG - 2) // 3, outer, 0)
        wait_u(_NG - 1, 1)
        issue_vadd(_NG - 1, 1)
        wait_vadd(_NG - 2, 0)
        compute(_NG - 2, 0)
        wait_vadd(_NG - 1, 1)
        compute(_NG - 1, 1)

        pltpu.sync_copy(out_v, out_hbm.at[pl.ds(base, _EPW)])

    return body(u_tab, vneg_tab, src, dst, ev)


def kernel(Eu, Ev, W1, b1, W2, b2, edge_index, edge_val):
    u = _transform(Eu, W1, b1, _mlp_block_pos)
    vneg = _transform(Ev, W2, b2, _mlp_block_neg)
    return _edge_values(u, vneg, edge_index[0], edge_index[1], edge_val)
